# baseline (speedup 1.0000x reference)
import functools

import jax
import jax.numpy as jnp
from jax.experimental import pallas as pl
from jax.experimental.pallas import tpu as pltpu


def _round_up(a, b):
    return (a + b - 1) // b * b


def _half(xh_ref, w_ref, xt_ref, slope, TH):
    xt_ref[...] = xh_ref[0].T
    even_t = xt_ref[pl.Slice(0, TH // 2, 2), :]
    odd_t = xt_ref[pl.Slice(1, TH // 2, 2), :]
    y_t = jnp.dot(even_t, w_ref[0], preferred_element_type=jnp.float32)
    y_t += jnp.dot(odd_t, w_ref[1], preferred_element_type=jnp.float32)
    y_t = jnp.where(y_t > 0, y_t, slope * y_t)
    return y_t.T


def _ds_ncl_kernel(xa_ref, xb_ref, w_ref, o_ref, xt_ref, *, slope, TH):
    o_ref[0, :, :TH // 2] = _half(xa_ref, w_ref, xt_ref, slope, TH)
    o_ref[0, :, TH // 2:] = _half(xb_ref, w_ref, xt_ref, slope, TH)


def kernel(x, w, *, slope=0.01):
    B, C, L = x.shape
    assert w.shape == (C, C, 2), w.shape
    Lout = L // 2
    x = x[:, :, :2 * Lout]

    Lp = _round_up(Lout, 8)
    if Lp != Lout:
        x = jnp.pad(x, ((0, 0), (0, 0), (0, 2 * (Lp - Lout))))
    TH = Lp

    w_t = jnp.transpose(w, (2, 1, 0))

    y = pl.pallas_call(
        functools.partial(_ds_ncl_kernel, slope=slope, TH=TH),
        out_shape=jax.ShapeDtypeStruct((B, C, Lp), x.dtype),
        grid=(B,),
        in_specs=[pl.BlockSpec((1, C, TH), lambda b: (b, 0, 0)),
                  pl.BlockSpec((1, C, TH), lambda b: (b, 0, 1)),
                  pl.BlockSpec((2, C, C), lambda b: (0, 0, 0))],
        out_specs=pl.BlockSpec((1, C, Lp), lambda b: (b, 0, 0)),
        scratch_shapes=[pltpu.VMEM((TH, C), jnp.float32)],
        compiler_params=pltpu.CompilerParams(
            dimension_semantics=("parallel",),
            vmem_limit_bytes=64 * 1024 * 1024),
    )(x, x, w_t)

    if Lp != Lout:
        y = y[:, :, :Lout]
    return y


# speedup vs baseline: 1.1687x; 1.1687x over previous
import functools

import jax
import jax.numpy as jnp
from jax.experimental import pallas as pl
from jax.experimental.pallas import tpu as pltpu


def _round_up(a, b):
    return (a + b - 1) // b * b


def _ds_ncl_kernel(x_ref, w_ref, o_ref, xt_ref, *, slope, BB):
    TO = o_ref.shape[2]
    for i in range(BB):
        xt_ref[...] = x_ref[i].T
        even_t = xt_ref[pl.Slice(0, TO, 2), :]
        odd_t = xt_ref[pl.Slice(1, TO, 2), :]
        y_t = jnp.dot(even_t, w_ref[0], preferred_element_type=jnp.float32)
        y_t += jnp.dot(odd_t, w_ref[1], preferred_element_type=jnp.float32)
        y_t = jnp.where(y_t > 0, y_t, slope * y_t)
        o_ref[i] = y_t.T.astype(o_ref.dtype)


def kernel(x, w, *, slope=0.01):
    B, C, L = x.shape
    assert w.shape == (C, C, 2), w.shape
    Lout = L // 2
    x = x[:, :, :2 * Lout]

    Lp = _round_up(Lout, 8)
    if Lp != Lout:
        x = jnp.pad(x, ((0, 0), (0, 0), (0, 2 * (Lp - Lout))))

    BB = 2 if B % 2 == 0 else 1

    w_t = jnp.transpose(w, (2, 1, 0))

    y = pl.pallas_call(
        functools.partial(_ds_ncl_kernel, slope=slope, BB=BB),
        out_shape=jax.ShapeDtypeStruct((B, C, Lp), x.dtype),
        grid=(B // BB,),
        in_specs=[pl.BlockSpec((BB, C, 2 * Lp), lambda b: (b, 0, 0)),
                  pl.BlockSpec((2, C, C), lambda b: (0, 0, 0))],
        out_specs=pl.BlockSpec((BB, C, Lp), lambda b: (b, 0, 0)),
        scratch_shapes=[pltpu.VMEM((2 * Lp, C), jnp.float32)],
        compiler_params=pltpu.CompilerParams(
            dimension_semantics=("parallel",),
            vmem_limit_bytes=64 * 1024 * 1024),
    )(x, w_t)

    if Lp != Lout:
        y = y[:, :, :Lout]
    return y
